# Initial kernel scaffold; baseline (speedup 1.0000x reference)
import jax
import jax.numpy as jnp
from jax.experimental import pallas as pl


def kernel(x, tables):
    raise NotImplementedError("write your pallas kernel here")


# speedup vs baseline: 10.9556x; 10.9556x over previous
import numpy as np
import jax
import jax.numpy as jnp
from jax import lax
from jax.experimental import pallas as pl
from jax.experimental.pallas import tpu as pltpu
from jax.experimental.pallas import tpu_sc as plsc

_F = 26
_V = 100000
_D = 16
_B = 4096
_NC, _NS, _L = 2, 16, 16
_NW = _NC * _NS
_BPW = _B // _NW
_P = (_F * (_F - 1)) // 2
_PPAD = 336
_CH = 112
_NCHUNK = (2 * _PPAD) // _CH

_ii_np, _jj_np = np.triu_indices(_F, k=1)
_iiP = np.zeros(_PPAD, np.int32)
_jjP = np.zeros(_PPAD, np.int32)
_iiP[:_P] = _ii_np
_jjP[:_P] = _jj_np


def _body(flat_hbm, idx_hbm, out_hbm, idx_v, rows_v, out2_v, sem):
    wid = lax.axis_index("s") * _NC + lax.axis_index("c")
    base = wid * _BPW
    lanes = jnp.arange(_L, dtype=jnp.int32)
    zeros = jnp.zeros((_L,), jnp.float32)

    def per_b(b, r):
        pltpu.sync_copy(idx_hbm.at[base + b], idx_v)
        copies = []
        for c in range(_NCHUNK):
            copies.append(pltpu.async_copy(
                flat_hbm.at[idx_v.at[c]],
                rows_v.at[pl.ds(c * _CH, _CH)], sem))
        for cp in copies:
            cp.wait()
        accs = [jnp.zeros((_L,), jnp.float32) for _ in range(8)]
        for p in range(_P):
            accs[p % 8] = accs[p % 8] + rows_v[p] * rows_v[_PPAD + p]
        acc = ((accs[0] + accs[1]) + (accs[2] + accs[3])) + (
            (accs[4] + accs[5]) + (accs[6] + accs[7]))
        s = jnp.sum(acc)
        lane = jnp.bitwise_and(b, _L - 1)
        r = jnp.where(lanes == lane, s, r)
        out2_v[lax.shift_right_logical(b, 4)] = r
        return jnp.where(lane == _L - 1, zeros, r)

    lax.fori_loop(0, _BPW, per_b, zeros)
    pltpu.sync_copy(out2_v, out_hbm.at[wid])


def _build_indices(x):
    jmul = jnp.asarray(_jjP * _V, dtype=jnp.int32)
    imul = jnp.asarray(_iiP * _V, dtype=jnp.int32)
    idx_a = x[:, _iiP].astype(jnp.int32) + jmul[None, :]
    idx_b = x[:, _jjP].astype(jnp.int32) + imul[None, :]
    return jnp.concatenate([idx_a, idx_b], axis=1).reshape(_B, _NCHUNK, _CH)


def kernel(x, tables):
    flat = tables.reshape(_F * _V, _D)
    idx = _build_indices(x)
    mesh = plsc.VectorSubcoreMesh(core_axis_name="c", subcore_axis_name="s",
                                  num_cores=_NC, num_subcores=_NS)
    out = pl.kernel(
        _body,
        out_type=jax.ShapeDtypeStruct((_NW, _BPW // _L, _L), jnp.float32),
        mesh=mesh,
        compiler_params=pltpu.CompilerParams(needs_layout_passes=False,
                                             use_tc_tiling_on_sc=False),
        scratch_types=[
            pltpu.VMEM((_NCHUNK, _CH), jnp.int32),
            pltpu.VMEM((2 * _PPAD, _D), jnp.float32),
            pltpu.VMEM((_BPW // _L, _L), jnp.float32),
            pltpu.SemaphoreType.DMA,
        ],
    )(flat, idx)
    return out.reshape(_B, 1)


# speedup vs baseline: 11.5996x; 1.0588x over previous
import numpy as np
import jax
import jax.numpy as jnp
from jax import lax
from jax.experimental import pallas as pl
from jax.experimental.pallas import tpu as pltpu
from jax.experimental.pallas import tpu_sc as plsc

_F = 26
_V = 100000
_D = 16
_B = 4096
_NC, _NS, _L = 2, 16, 16
_NW = _NC * _NS
_BPW = _B // _NW
_P = (_F * (_F - 1)) // 2
_PPAD = 336
_CH = 112
_NCHUNK = (2 * _PPAD) // _CH

_ii_np, _jj_np = np.triu_indices(_F, k=1)
_iiP = np.zeros(_PPAD, np.int32)
_jjP = np.zeros(_PPAD, np.int32)
_iiP[:_P] = _ii_np
_jjP[:_P] = _jj_np


def _body(flat_hbm, idx_hbm, out_hbm, idx_v, rows_a, rows_b, out2_v,
          sem_a, sem_b):
    wid = lax.axis_index("s") * _NC + lax.axis_index("c")
    lanes = jnp.arange(_L, dtype=jnp.int32)
    zeros = jnp.zeros((_L,), jnp.float32)

    def fire(n, rows_ref, sem):
        ib = idx_v.at[n]
        for c in range(_NCHUNK):
            pltpu.async_copy(flat_hbm.at[ib.at[c]],
                             rows_ref.at[pl.ds(c * _CH, _CH)], sem)

    def drain(rows_ref, sem):
        pltpu.make_async_copy(flat_hbm.at[pl.ds(0, 2 * _PPAD)],
                              rows_ref, sem).wait()

    def compute(rows_ref):
        accs = [jnp.zeros((_L,), jnp.float32) for _ in range(8)]
        for p in range(_P):
            accs[p % 8] = accs[p % 8] + rows_ref[p] * rows_ref[_PPAD + p]
        acc = ((accs[0] + accs[1]) + (accs[2] + accs[3])) + (
            (accs[4] + accs[5]) + (accs[6] + accs[7]))
        return jnp.sum(acc)

    pltpu.sync_copy(idx_hbm.at[wid], idx_v)
    fire(0, rows_a, sem_a)
    fire(1, rows_b, sem_b)

    def step(k, r):
        def half(b, nxt, rows_ref, sem, r):
            drain(rows_ref, sem)
            s = compute(rows_ref)
            lane = jnp.bitwise_and(b, _L - 1)
            r = jnp.where(lanes == lane, s, r)
            out2_v[lax.shift_right_logical(b, 4)] = r

            @pl.when(k < _BPW // 2 - 1)
            def _():
                fire(nxt, rows_ref, sem)

            return jnp.where(lane == _L - 1, zeros, r)

        b0 = 2 * k
        r = half(b0, b0 + 2, rows_a, sem_a, r)
        r = half(b0 + 1, b0 + 3, rows_b, sem_b, r)
        return r

    lax.fori_loop(0, _BPW // 2, step, zeros)
    pltpu.sync_copy(out2_v, out_hbm.at[wid])


def _build_indices(x):
    jmul = jnp.asarray(_jjP * _V, dtype=jnp.int32)
    imul = jnp.asarray(_iiP * _V, dtype=jnp.int32)
    idx_a = x[:, _iiP].astype(jnp.int32) + jmul[None, :]
    idx_b = x[:, _jjP].astype(jnp.int32) + imul[None, :]
    return jnp.concatenate([idx_a, idx_b], axis=1).reshape(
        _NW, _BPW, _NCHUNK, _CH)


def kernel(x, tables):
    flat = tables.reshape(_F * _V, _D)
    idx = _build_indices(x)
    mesh = plsc.VectorSubcoreMesh(core_axis_name="c", subcore_axis_name="s",
                                  num_cores=_NC, num_subcores=_NS)
    out = pl.kernel(
        _body,
        out_type=jax.ShapeDtypeStruct((_NW, _BPW // _L, _L), jnp.float32),
        mesh=mesh,
        compiler_params=pltpu.CompilerParams(needs_layout_passes=False,
                                             use_tc_tiling_on_sc=False),
        scratch_types=[
            pltpu.VMEM((_BPW, _NCHUNK, _CH), jnp.int32),
            pltpu.VMEM((2 * _PPAD, _D), jnp.float32),
            pltpu.VMEM((2 * _PPAD, _D), jnp.float32),
            pltpu.VMEM((_BPW // _L, _L), jnp.float32),
            pltpu.SemaphoreType.DMA,
            pltpu.SemaphoreType.DMA,
        ],
    )(flat, idx)
    return out.reshape(_B, 1)


# speedup vs baseline: 11.7331x; 1.0115x over previous
import numpy as np
import jax
import jax.numpy as jnp
from jax import lax
from jax.experimental import pallas as pl
from jax.experimental.pallas import tpu as pltpu
from jax.experimental.pallas import tpu_sc as plsc

_F = 26
_V = 100000
_D = 16
_B = 4096
_NC, _NS, _L = 2, 16, 16
_NW = _NC * _NS
_BPW = _B // _NW
_P = (_F * (_F - 1)) // 2
_PPAD = 336
_CH = 112
_NCHUNK = (2 * _PPAD) // _CH

_ii_np, _jj_np = np.triu_indices(_F, k=1)
_iiP = np.zeros(_PPAD, np.int32)
_jjP = np.zeros(_PPAD, np.int32)
_iiP[:_P] = _ii_np
_jjP[:_P] = _jj_np


def _body(flat_hbm, idx_hbm, out_hbm, idx_v, rows_a, rows_b, out2_v,
          sem_a, sem_b):
    wid = lax.axis_index("s") * _NC + lax.axis_index("c")
    lanes = jnp.arange(_L, dtype=jnp.int32)
    zeros = jnp.zeros((_L,), jnp.float32)

    def fire(n, rows_ref, sem):
        for c in range(_NCHUNK):
            pltpu.async_copy(
                flat_hbm.at[idx_v.at[pl.ds(n * (2 * _PPAD) + c * _CH, _CH)]],
                rows_ref.at[pl.ds(c * _CH, _CH)], sem)

    def drain(rows_ref, sem):
        pltpu.make_async_copy(flat_hbm.at[pl.ds(0, 2 * _PPAD)],
                              rows_ref, sem).wait()

    def compute(rows_ref):
        accs = [jnp.zeros((_L,), jnp.float32) for _ in range(8)]
        for p in range(_P):
            accs[p % 8] = accs[p % 8] + rows_ref[p] * rows_ref[_PPAD + p]
        acc = ((accs[0] + accs[1]) + (accs[2] + accs[3])) + (
            (accs[4] + accs[5]) + (accs[6] + accs[7]))
        return jnp.sum(acc)

    pltpu.sync_copy(idx_hbm.at[wid], idx_v)
    fire(0, rows_a, sem_a)
    fire(1, rows_b, sem_b)

    def step(k, r):
        def half(b, nxt, rows_ref, sem, r):
            drain(rows_ref, sem)
            s = compute(rows_ref)
            lane = jnp.bitwise_and(b, _L - 1)
            r = jnp.where(lanes == lane, s, r)
            out2_v[lax.shift_right_logical(b, 4)] = r

            @pl.when(k < _BPW // 2 - 1)
            def _():
                fire(nxt, rows_ref, sem)

            return jnp.where(lane == _L - 1, zeros, r)

        b0 = 2 * k
        r = half(b0, b0 + 2, rows_a, sem_a, r)
        r = half(b0 + 1, b0 + 3, rows_b, sem_b, r)
        return r

    lax.fori_loop(0, _BPW // 2, step, zeros)
    pltpu.sync_copy(out2_v, out_hbm.at[wid])


def _build_indices(x):
    jmul = jnp.asarray(_jjP * _V, dtype=jnp.int32)
    imul = jnp.asarray(_iiP * _V, dtype=jnp.int32)
    idx_a = x[:, _iiP].astype(jnp.int32) + jmul[None, :]
    idx_b = x[:, _jjP].astype(jnp.int32) + imul[None, :]
    return jnp.concatenate([idx_a, idx_b], axis=1).reshape(
        _NW, _BPW * 2 * _PPAD)


def kernel(x, tables):
    flat = tables.reshape(_F * _V, _D)
    idx = _build_indices(x)
    mesh = plsc.VectorSubcoreMesh(core_axis_name="c", subcore_axis_name="s",
                                  num_cores=_NC, num_subcores=_NS)
    out = pl.kernel(
        _body,
        out_type=jax.ShapeDtypeStruct((_NW, _BPW // _L, _L), jnp.float32),
        mesh=mesh,
        compiler_params=pltpu.CompilerParams(needs_layout_passes=False,
                                             use_tc_tiling_on_sc=False),
        scratch_types=[
            pltpu.VMEM((_BPW * 2 * _PPAD,), jnp.int32),
            pltpu.VMEM((2 * _PPAD, _D), jnp.float32),
            pltpu.VMEM((2 * _PPAD, _D), jnp.float32),
            pltpu.VMEM((_BPW // _L, _L), jnp.float32),
            pltpu.SemaphoreType.DMA,
            pltpu.SemaphoreType.DMA,
        ],
    )(flat, idx)
    return out.reshape(_B, 1)


# speedup vs baseline: 12.3308x; 1.0509x over previous
import numpy as np
import jax
import jax.numpy as jnp
from jax import lax
from jax.experimental import pallas as pl
from jax.experimental.pallas import tpu as pltpu
from jax.experimental.pallas import tpu_sc as plsc

_F = 26
_V = 100000
_D = 16
_B = 4096
_NC, _NS, _L = 2, 16, 16
_NW = _NC * _NS
_BPW = _B // _NW
_P = (_F * (_F - 1)) // 2
_XP = 32
_R = _F * _F


def _body(tab_hbm, xp_hbm, out_hbm, x_v, rows_a, rows_b, out2_v,
          sem_a, sem_b):
    wid = lax.axis_index("s") * _NC + lax.axis_index("c")
    lanes = jnp.arange(_L, dtype=jnp.int32)
    zeros = jnp.zeros((_L,), jnp.float32)

    def fire(n, rows_ref, sem):
        ix = x_v.at[pl.ds(n * _XP, _F)]
        for t in range(_F):
            pltpu.async_copy(tab_hbm.at[t].at[ix],
                             rows_ref.at[pl.ds(t * _F, _F)], sem)

    def drain(rows_ref, sem):
        pltpu.make_async_copy(tab_hbm.at[0].at[pl.ds(0, _R)],
                              rows_ref, sem).wait()

    def compute(rows_ref):
        accs = [jnp.zeros((_L,), jnp.float32) for _ in range(8)]
        k = 0
        for i in range(_F):
            for j in range(i + 1, _F):
                accs[k % 8] = accs[k % 8] + (
                    rows_ref[i * _F + j] * rows_ref[j * _F + i])
                k += 1
        acc = ((accs[0] + accs[1]) + (accs[2] + accs[3])) + (
            (accs[4] + accs[5]) + (accs[6] + accs[7]))
        return jnp.sum(acc)

    pltpu.sync_copy(xp_hbm.at[pl.ds(wid * _BPW * _XP, _BPW * _XP)], x_v)
    fire(0, rows_a, sem_a)
    fire(1, rows_b, sem_b)

    def step(k, r):
        def half(b, nxt, rows_ref, sem, r):
            drain(rows_ref, sem)
            s = compute(rows_ref)
            lane = jnp.bitwise_and(b, _L - 1)
            r = jnp.where(lanes == lane, s, r)
            out2_v[lax.shift_right_logical(b, 4)] = r

            @pl.when(k < _BPW // 2 - 1)
            def _():
                fire(nxt, rows_ref, sem)

            return jnp.where(lane == _L - 1, zeros, r)

        b0 = 2 * k
        r = half(b0, b0 + 2, rows_a, sem_a, r)
        r = half(b0 + 1, b0 + 3, rows_b, sem_b, r)
        return r

    lax.fori_loop(0, _BPW // 2, step, zeros)
    pltpu.sync_copy(out2_v, out_hbm.at[wid])


def kernel(x, tables):
    xp = jnp.pad(x.astype(jnp.int32), ((0, 0), (0, _XP - _F))).reshape(-1)
    mesh = plsc.VectorSubcoreMesh(core_axis_name="c", subcore_axis_name="s",
                                  num_cores=_NC, num_subcores=_NS)
    out = pl.kernel(
        _body,
        out_type=jax.ShapeDtypeStruct((_NW, _BPW // _L, _L), jnp.float32),
        mesh=mesh,
        compiler_params=pltpu.CompilerParams(needs_layout_passes=False,
                                             use_tc_tiling_on_sc=False),
        scratch_types=[
            pltpu.VMEM((_BPW * _XP,), jnp.int32),
            pltpu.VMEM((_R, _D), jnp.float32),
            pltpu.VMEM((_R, _D), jnp.float32),
            pltpu.VMEM((_BPW // _L, _L), jnp.float32),
            pltpu.SemaphoreType.DMA,
            pltpu.SemaphoreType.DMA,
        ],
    )(tables, xp)
    return out.reshape(_B, 1)


# speedup vs baseline: 27.6810x; 2.2449x over previous
import numpy as np
import jax
import jax.numpy as jnp
from jax import lax
from jax.experimental import pallas as pl
from jax.experimental.pallas import tpu as pltpu
from jax.experimental.pallas import tpu_sc as plsc

_F = 26
_V = 100000
_D = 16
_B = 4096
_NC, _NS, _L = 2, 16, 16
_NW = _NC * _NS
_BPW = _B // _NW
_P = (_F * (_F - 1)) // 2
_XP = 32
_W = 512
_VC = 2048


def _tbody(x_ref, o_ref):
    x = x_ref[...].reshape(_F * _D, _VC)
    o_ref[:, 0:_F * _D] = jnp.transpose(x, (1, 0))


def _transpose_tables(tables):
    tview = jnp.transpose(tables, (0, 2, 1))
    return pl.pallas_call(
        _tbody,
        grid=((_V + _VC - 1) // _VC,),
        in_specs=[pl.BlockSpec((_F, _D, _VC), lambda vc: (0, 0, vc))],
        out_specs=pl.BlockSpec((_VC, _W), lambda vc: (vc, 0)),
        out_shape=jax.ShapeDtypeStruct((_V, _W), jnp.float32),
    )(tview)


def _body(z_hbm, xp_hbm, out_hbm, x_v, rows_a, rows_b, out2_v,
          sem_a, sem_b):
    wid = lax.axis_index("s") * _NC + lax.axis_index("c")
    lanes = jnp.arange(_L, dtype=jnp.int32)
    zeros = jnp.zeros((_L,), jnp.float32)

    def fire(n, rows_ref, sem):
        ix = x_v.at[pl.ds(n * _XP, _F)]
        pltpu.async_copy(z_hbm.at[ix], rows_ref, sem)

    def drain(rows_ref, sem):
        pltpu.make_async_copy(z_hbm.at[pl.ds(0, _F)], rows_ref, sem).wait()

    def compute(rows_ref):
        accs = [jnp.zeros((_L,), jnp.float32) for _ in range(8)]
        k = 0
        for i in range(_F):
            for j in range(i + 1, _F):
                accs[k % 8] = accs[k % 8] + (
                    rows_ref[j, pl.ds(i * _D, _D)]
                    * rows_ref[i, pl.ds(j * _D, _D)])
                k += 1
        acc = ((accs[0] + accs[1]) + (accs[2] + accs[3])) + (
            (accs[4] + accs[5]) + (accs[6] + accs[7]))
        return jnp.sum(acc)

    pltpu.sync_copy(xp_hbm.at[pl.ds(wid * _BPW * _XP, _BPW * _XP)], x_v)
    fire(0, rows_a, sem_a)
    fire(1, rows_b, sem_b)

    def step(k, r):
        def half(b, nxt, rows_ref, sem, r):
            drain(rows_ref, sem)
            s = compute(rows_ref)
            lane = jnp.bitwise_and(b, _L - 1)
            r = jnp.where(lanes == lane, s, r)
            out2_v[lax.shift_right_logical(b, 4)] = r

            @pl.when(k < _BPW // 2 - 1)
            def _():
                fire(nxt, rows_ref, sem)

            return jnp.where(lane == _L - 1, zeros, r)

        b0 = 2 * k
        r = half(b0, b0 + 2, rows_a, sem_a, r)
        r = half(b0 + 1, b0 + 3, rows_b, sem_b, r)
        return r

    lax.fori_loop(0, _BPW // 2, step, zeros)
    pltpu.sync_copy(out2_v, out_hbm.at[wid])


def kernel(x, tables):
    z = _transpose_tables(tables)
    xp = jnp.pad(x.astype(jnp.int32), ((0, 0), (0, _XP - _F))).reshape(-1)
    mesh = plsc.VectorSubcoreMesh(core_axis_name="c", subcore_axis_name="s",
                                  num_cores=_NC, num_subcores=_NS)
    out = pl.kernel(
        _body,
        out_type=jax.ShapeDtypeStruct((_NW, _BPW // _L, _L), jnp.float32),
        mesh=mesh,
        compiler_params=pltpu.CompilerParams(needs_layout_passes=False,
                                             use_tc_tiling_on_sc=False),
        scratch_types=[
            pltpu.VMEM((_BPW * _XP,), jnp.int32),
            pltpu.VMEM((_F, _W), jnp.float32),
            pltpu.VMEM((_F, _W), jnp.float32),
            pltpu.VMEM((_BPW // _L, _L), jnp.float32),
            pltpu.SemaphoreType.DMA,
            pltpu.SemaphoreType.DMA,
        ],
    )(z, xp)
    return out.reshape(_B, 1)
